# Initial kernel scaffold; baseline (speedup 1.0000x reference)
import functools

import jax
import jax.numpy as jnp
from jax import lax
from jax.experimental import pallas as pl
from jax.experimental.pallas import tpu as pltpu
from jax.experimental.pallas import tpu_sc as plsc

BATCH = 16384
EMB = 32
NC = 2
NS = 16
NW = NC * NS
B_PER_W = BATCH // NW
CHUNK = 128
NCHUNK = B_PER_W // CHUNK

_sc_mesh = plsc.VectorSubcoreMesh(core_axis_name="c", subcore_axis_name="s")


@functools.partial(
    pl.kernel,
    mesh=_sc_mesh,
    out_type=[
        jax.ShapeDtypeStruct((BATCH, EMB), jnp.float32),
        jax.ShapeDtypeStruct((BATCH, EMB), jnp.float32),
    ],
    scratch_types=[
        pltpu.VMEM((NCHUNK, CHUNK), jnp.int32),
        pltpu.VMEM((NCHUNK, CHUNK), jnp.int32),
        pltpu.VMEM((B_PER_W, EMB), jnp.float32),
        pltpu.VMEM((B_PER_W, EMB), jnp.float32),
        pltpu.SemaphoreType.DMA,
        pltpu.SemaphoreType.DMA,
    ],
)
def _sc_gather(item_tbl, item_idx, user_tbl, user_idx,
               item_out, user_out,
               iidx_v, uidx_v, irows_v, urows_v, isem, usem):
    wid = lax.axis_index("s") * NC + lax.axis_index("c")
    base = wid * B_PER_W
    pltpu.sync_copy(item_idx.at[wid], iidx_v)
    pltpu.sync_copy(user_idx.at[wid], uidx_v)
    copies = []
    for j in range(NCHUNK):
        copies.append(pltpu.async_copy(
            item_tbl.at[iidx_v.at[j]],
            irows_v.at[pl.ds(j * CHUNK, CHUNK)], isem))
        copies.append(pltpu.async_copy(
            user_tbl.at[uidx_v.at[j]],
            urows_v.at[pl.ds(j * CHUNK, CHUNK)], usem))
    for c in copies:
        c.wait()
    pltpu.sync_copy(irows_v, item_out.at[pl.ds(base, B_PER_W)])
    pltpu.sync_copy(urows_v, user_out.at[pl.ds(base, B_PER_W)])


BLK = 2048


def _mlp_body(item_ref, user_ref, w1a_ref, w1b_ref, b1_ref,
              w2_ref, b2_ref, w3_ref, b3_ref, w4_ref, b4_ref, out_ref):
    h = item_ref[...] @ w1a_ref[...] + user_ref[...] @ w1b_ref[...]
    h = jax.nn.relu(h + b1_ref[...])
    h = jax.nn.relu(h @ w2_ref[...] + b2_ref[...])
    h = jax.nn.relu(h @ w3_ref[...] + b3_ref[...])
    o = h @ w4_ref[...] + b4_ref[...]
    out_ref[...] = jax.nn.sigmoid(o)


def _mlp(item_rows, user_rows, w1a, w1b, b1, w2, b2, w3, b3, w4, b4):
    grid = (BATCH // BLK,)
    full = lambda shape: pl.BlockSpec(shape, lambda i: (0, 0))
    return pl.pallas_call(
        _mlp_body,
        grid=grid,
        in_specs=[
            pl.BlockSpec((BLK, EMB), lambda i: (i, 0)),
            pl.BlockSpec((BLK, EMB), lambda i: (i, 0)),
            full(w1a.shape), full(w1b.shape), full(b1.shape),
            full(w2.shape), full(b2.shape),
            full(w3.shape), full(b3.shape),
            full(w4.shape), full(b4.shape),
        ],
        out_specs=pl.BlockSpec((BLK, 1), lambda i: (i, 0)),
        out_shape=jax.ShapeDtypeStruct((BATCH, 1), jnp.float32),
    )(item_rows, user_rows, w1a, w1b, b1, w2, b2, w3, b3, w4, b4)


def kernel(item_input, user_input, emb_item, emb_user,
           W1, b1, W2, b2, W3, b3, W4, b4):
    item_idx = item_input.astype(jnp.int32).reshape(NW, NCHUNK, CHUNK)
    user_idx = user_input.astype(jnp.int32).reshape(NW, NCHUNK, CHUNK)
    item_rows, user_rows = _sc_gather(emb_item, item_idx, emb_user, user_idx)
    w1a = W1[:, :EMB].T
    w1b = W1[:, EMB:].T
    out = _mlp(item_rows, user_rows,
               w1a, w1b, b1.reshape(1, -1),
               W2.T, b2.reshape(1, -1),
               W3.T, b3.reshape(1, -1),
               W4.T, b4.reshape(1, 1))
    return out.reshape(BATCH)


# speedup vs baseline: 7.0101x; 7.0101x over previous
import functools

import jax
import jax.numpy as jnp
from jax import lax
from jax.experimental import pallas as pl
from jax.experimental.pallas import tpu as pltpu

BATCH = 16384
EMB = 32
BLK = 2048


def _mlp_body(item_ref, user_ref, w1a_ref, w1b_ref, b1_ref,
              w2_ref, b2_ref, w3_ref, b3_ref, w4_ref, b4_ref, out_ref):
    h = item_ref[...] @ w1a_ref[...] + user_ref[...] @ w1b_ref[...]
    h = jax.nn.relu(h + b1_ref[...])
    h = jax.nn.relu(h @ w2_ref[...] + b2_ref[...])
    h = jax.nn.relu(h @ w3_ref[...] + b3_ref[...])
    o = h @ w4_ref[...] + b4_ref[...]
    out_ref[...] = jax.nn.sigmoid(o)


def _mlp(item_rows, user_rows, w1a, w1b, b1, w2, b2, w3, b3, w4, b4):
    grid = (BATCH // BLK,)
    full = lambda shape: pl.BlockSpec(shape, lambda i: (0, 0))
    return pl.pallas_call(
        _mlp_body,
        grid=grid,
        in_specs=[
            pl.BlockSpec((BLK, EMB), lambda i: (i, 0)),
            pl.BlockSpec((BLK, EMB), lambda i: (i, 0)),
            full(w1a.shape), full(w1b.shape), full(b1.shape),
            full(w2.shape), full(b2.shape),
            full(w3.shape), full(b3.shape),
            full(w4.shape), full(b4.shape),
        ],
        out_specs=pl.BlockSpec((BLK, 1), lambda i: (i, 0)),
        out_shape=jax.ShapeDtypeStruct((BATCH, 1), jnp.float32),
    )(item_rows, user_rows, w1a, w1b, b1, w2, b2, w3, b3, w4, b4)


def kernel(item_input, user_input, emb_item, emb_user,
           W1, b1, W2, b2, W3, b3, W4, b4):
    item_rows = jnp.take(emb_item, item_input, axis=0)
    user_rows = jnp.take(emb_user, user_input, axis=0)
    out = _mlp(item_rows, user_rows,
               W1[:, :EMB].T, W1[:, EMB:].T, b1.reshape(1, -1),
               W2.T, b2.reshape(1, -1),
               W3.T, b3.reshape(1, -1),
               W4.T, b4.reshape(1, 1))
    return out.reshape(BATCH)
